# baseline (speedup 1.0000x reference)
import jax
import jax.numpy as jnp
from jax.experimental import pallas as pl
from jax.experimental.pallas import tpu as pltpu

_TM = 200


def _support_kernel(x_ref, w_ref, o_ref):
    o_ref[...] = jnp.dot(x_ref[...], w_ref[...],
                         preferred_element_type=jnp.float32)


def _gcn_kernel(adj_ref, s_ref, b_ref, o_ref):
    o_ref[...] = jnp.dot(adj_ref[...], s_ref[...],
                         preferred_element_type=jnp.float32) + b_ref[...]


def kernel(x, adj, W, b):
    n, nfeat = x.shape
    nclass = W.shape[1]

    support = pl.pallas_call(
        _support_kernel,
        grid=(5,),
        in_specs=[
            pl.BlockSpec((n // 5, nfeat), lambda i: (i, 0)),
            pl.BlockSpec((nfeat, nclass), lambda i: (0, 0)),
        ],
        out_specs=pl.BlockSpec((n // 5, nclass), lambda i: (i, 0)),
        out_shape=jax.ShapeDtypeStruct((n, nclass), jnp.float32),
    )(x, W)

    nm = n // _TM
    out = pl.pallas_call(
        _gcn_kernel,
        grid=(nm,),
        in_specs=[
            pl.BlockSpec((_TM, n), lambda m: (m, 0)),
            pl.BlockSpec((n, nclass), lambda m: (0, 0)),
            pl.BlockSpec((1, nclass), lambda m: (0, 0)),
        ],
        out_specs=pl.BlockSpec((_TM, nclass), lambda m: (m, 0)),
        out_shape=jax.ShapeDtypeStruct((n, nclass), jnp.float32),
        compiler_params=pltpu.CompilerParams(
            dimension_semantics=("parallel",),
        ),
    )(adj, support, b.reshape(1, nclass))
    return out


# speedup vs baseline: 1.0004x; 1.0004x over previous
import jax
import jax.numpy as jnp
from jax.experimental import pallas as pl
from jax.experimental.pallas import tpu as pltpu

_TM = 200


def _support_kernel(x_ref, w_ref, o_ref):
    o_ref[...] = jnp.dot(x_ref[...], w_ref[...],
                         preferred_element_type=jnp.float32
                         ).astype(jnp.bfloat16)


def _gcn_kernel(adj_ref, s_ref, b_ref, o_ref):
    o_ref[...] = jnp.dot(adj_ref[...].astype(jnp.bfloat16), s_ref[...],
                         preferred_element_type=jnp.float32) + b_ref[...]


def kernel(x, adj, W, b):
    n, nfeat = x.shape
    nclass = W.shape[1]

    support = pl.pallas_call(
        _support_kernel,
        grid=(5,),
        in_specs=[
            pl.BlockSpec((n // 5, nfeat), lambda i: (i, 0)),
            pl.BlockSpec((nfeat, nclass), lambda i: (0, 0)),
        ],
        out_specs=pl.BlockSpec((n // 5, nclass), lambda i: (i, 0)),
        out_shape=jax.ShapeDtypeStruct((n, nclass), jnp.bfloat16),
    )(x, W)

    nm = n // _TM
    out = pl.pallas_call(
        _gcn_kernel,
        grid=(nm,),
        in_specs=[
            pl.BlockSpec((_TM, n), lambda m: (m, 0)),
            pl.BlockSpec((n, nclass), lambda m: (0, 0)),
            pl.BlockSpec((1, nclass), lambda m: (0, 0)),
        ],
        out_specs=pl.BlockSpec((_TM, nclass), lambda m: (m, 0)),
        out_shape=jax.ShapeDtypeStruct((n, nclass), jnp.float32),
        compiler_params=pltpu.CompilerParams(
            dimension_semantics=("parallel",),
        ),
    )(adj, support, b.reshape(1, nclass))
    return out
